# baseline (speedup 1.0000x reference)
import functools

import jax
import jax.numpy as jnp
from jax import lax
from jax.experimental import pallas as pl
from jax.experimental.pallas import tpu as pltpu
from jax.experimental.pallas import tpu_sc as plsc

K, D, N, M = 16, 256, 8192, 256
L = 16
NW = 32
D_PER_W = D * K // NW
GROUPS = N // L
UNROLL = 2
R = 4
RT = 4
MAIN = 15
DT0 = 2 * R * MAIN
DT1 = DT0 + RT


def _sc_body(feat_hbm, label_hbm, clab_hbm, outf_hbm, outs_hbm,
             idx_v, lab_v, buf_a, buf_b, cnt_v, lsum_v, scale_v,
             acc0_v, acc1_v, acc2_v, acc3_v, acc4_v, acc5_v,
             obuf_a, obuf_b, sem_a, sem_b, osem_a, osem_b, ssem):
    wid = lax.axis_index("s") * 2 + lax.axis_index("c")
    k = wid // 2
    dbase = (wid % 2) * D_PER_W

    ones = jnp.ones((L,), jnp.float32)
    zeros = jnp.zeros((L,), jnp.float32)
    accs = (acc0_v, acc1_v, acc2_v, acc3_v, acc4_v, acc5_v)

    cp_idx = pltpu.async_copy(clab_hbm.at[k], idx_v, ssem)
    cp_lab = pltpu.async_copy(label_hbm.at[k], lab_v, ssem)
    pltpu.async_copy(feat_hbm.at[k, pl.ds(dbase, R)], buf_a, sem_a)
    cp_idx.wait()
    cp_lab.wait()

    for j in range(M // L):
        cnt_v[pl.ds(j * L, L)] = zeros
        lsum_v[pl.ds(j * L, L)] = zeros

    @plsc.parallel_loop(0, GROUPS, 1, unroll=UNROLL)
    def _hist(i):
        sl = pl.ds(i * L, L)
        iv = idx_v[sl]
        plsc.addupdate_scatter(cnt_v, [iv], ones)
        plsc.addupdate_scatter(lsum_v, [iv], lab_v[sl])

    for j in range(M // L):
        sl = pl.ds(j * L, L)
        size = cnt_v[sl]
        denom = jnp.maximum(size, 1.0)
        mean_lab = lsum_v[sl] / denom
        mask = jnp.where(mean_lab > 0.5, 1.0, 0.0).astype(jnp.float32)
        scale_v[sl] = mask / denom
        obuf_a[0, sl] = size * mask

    @pl.when(dbase == 0)
    def _():
        pltpu.sync_copy(obuf_a.at[0], outs_hbm.at[k])

    def process(buf, obuf, osem, d, rows, first, prev_d, prev_rows):
        for r in range(rows):
            for j in range(M // L):
                accs[r][pl.ds(j * L, L)] = zeros

        @plsc.parallel_loop(0, GROUPS, 1, unroll=UNROLL)
        def _seg(i):
            sl = pl.ds(i * L, L)
            iv = idx_v[sl]
            for r in range(rows):
                plsc.addupdate_scatter(accs[r], [iv], buf[r, sl])

        @pl.when(jnp.logical_not(first))
        def _():
            pltpu.make_async_copy(
                obuf.at[0:prev_rows],
                outf_hbm.at[k, pl.ds(prev_d, prev_rows)], osem).wait()

        for r in range(rows):
            for j in range(M // L):
                sl = pl.ds(j * L, L)
                obuf[r, sl] = accs[r][sl] * scale_v[sl]
        pltpu.async_copy(
            obuf.at[0:rows], outf_hbm.at[k, pl.ds(d, rows)], osem)

    def chunk_step(j, _):
        d0 = dbase + 2 * R * j
        d1 = d0 + R
        pltpu.make_async_copy(feat_hbm.at[k, pl.ds(d0, R)], buf_a, sem_a).wait()
        pltpu.async_copy(feat_hbm.at[k, pl.ds(d1, R)], buf_b, sem_b)
        process(buf_a, obuf_a, osem_a, d0, R, j == 0, d0 - 2 * R, R)
        pltpu.make_async_copy(feat_hbm.at[k, pl.ds(d1, R)], buf_b, sem_b).wait()

        @pl.when(j < MAIN - 1)
        def _():
            pltpu.async_copy(feat_hbm.at[k, pl.ds(d1 + R, R)], buf_a, sem_a)

        @pl.when(j == MAIN - 1)
        def _():
            pltpu.async_copy(
                feat_hbm.at[k, pl.ds(dbase + DT0, RT)], buf_a.at[0:RT], sem_a)

        process(buf_b, obuf_b, osem_b, d1, R, j == 0, d1 - 2 * R, R)
        return 0

    lax.fori_loop(0, MAIN, chunk_step, 0)

    dt0 = dbase + DT0
    dt1 = dbase + DT1
    pltpu.make_async_copy(
        feat_hbm.at[k, pl.ds(dt0, RT)], buf_a.at[0:RT], sem_a).wait()
    pltpu.async_copy(feat_hbm.at[k, pl.ds(dt1, RT)], buf_b.at[0:RT], sem_b)
    process(buf_a, obuf_a, osem_a, dt0, RT, False, dt0 - 2 * R, R)
    pltpu.make_async_copy(
        feat_hbm.at[k, pl.ds(dt1, RT)], buf_b.at[0:RT], sem_b).wait()
    process(buf_b, obuf_b, osem_b, dt1, RT, False, dt1 - 2 * R, R)

    pltpu.make_async_copy(
        obuf_a.at[0:RT], outf_hbm.at[k, pl.ds(dt0, RT)], osem_a).wait()
    pltpu.make_async_copy(
        obuf_b.at[0:RT], outf_hbm.at[k, pl.ds(dt1, RT)], osem_b).wait()


@jax.jit
def kernel(feat, label, cluster_label):
    mesh = plsc.VectorSubcoreMesh(core_axis_name="c", subcore_axis_name="s")
    fn = functools.partial(
        pl.kernel,
        out_type=[
            jax.ShapeDtypeStruct((K, D, M), jnp.float32),
            jax.ShapeDtypeStruct((K, M), jnp.float32),
        ],
        mesh=mesh,
        compiler_params=pltpu.CompilerParams(
            needs_layout_passes=False,
            disable_bounds_checks=True,
            use_tc_tiling_on_sc=False,
        ),
        scratch_types=[
            pltpu.VMEM((N,), jnp.int32),
            pltpu.VMEM((N,), jnp.float32),
            pltpu.VMEM((R, N), jnp.float32),
            pltpu.VMEM((R, N), jnp.float32),
            pltpu.VMEM((M,), jnp.float32),
            pltpu.VMEM((M,), jnp.float32),
            pltpu.VMEM((M,), jnp.float32),
            pltpu.VMEM((M,), jnp.float32),
            pltpu.VMEM((M,), jnp.float32),
            pltpu.VMEM((M,), jnp.float32),
            pltpu.VMEM((M,), jnp.float32),
            pltpu.VMEM((M,), jnp.float32),
            pltpu.VMEM((M,), jnp.float32),
            pltpu.VMEM((R, M), jnp.float32),
            pltpu.VMEM((R, M), jnp.float32),
            pltpu.SemaphoreType.DMA,
            pltpu.SemaphoreType.DMA,
            pltpu.SemaphoreType.DMA,
            pltpu.SemaphoreType.DMA,
            pltpu.SemaphoreType.DMA,
        ],
    )(_sc_body)
    outf, outs = fn(feat, label, cluster_label)
    return outf, outs


# speedup vs baseline: 1.6147x; 1.6147x over previous
import functools

import jax
import jax.numpy as jnp
from jax import lax
from jax.experimental import pallas as pl
from jax.experimental.pallas import tpu as pltpu
from jax.experimental.pallas import tpu_sc as plsc

K, D, N, M = 16, 256, 8192, 256
L = 16
NW = 32
D_PER_W = D * K // NW
GROUPS = N // L
UNROLL = 2
R = 4
QUADS = D_PER_W // R
HALF = QUADS // 2


def _sc_body(feat_hbm, label_hbm, clab_hbm, outf_hbm, outs_hbm,
             idx_v, lab_v, buf_a, buf_b, cnt_v, lsum_v, scale_v,
             acc0_v, acc1_v, acc2_v, acc3_v, obuf_a, obuf_b,
             sem_a, sem_b, osem_a, osem_b, ssem):
    wid = lax.axis_index("s") * 2 + lax.axis_index("c")
    k = wid // 2
    dbase = (wid % 2) * D_PER_W

    ones = jnp.ones((L,), jnp.float32)
    zeros = jnp.zeros((L,), jnp.float32)

    cp_idx = pltpu.async_copy(clab_hbm.at[k], idx_v, ssem)
    cp_lab = pltpu.async_copy(label_hbm.at[k], lab_v, ssem)
    pltpu.async_copy(feat_hbm.at[k, pl.ds(dbase, R)], buf_a, sem_a)
    cp_idx.wait()
    cp_lab.wait()

    for j in range(M // L):
        cnt_v[pl.ds(j * L, L)] = zeros
        lsum_v[pl.ds(j * L, L)] = zeros

    @plsc.parallel_loop(0, GROUPS, 1, unroll=UNROLL)
    def _hist(i):
        sl = pl.ds(i * L, L)
        iv = idx_v[sl]
        plsc.addupdate_scatter(cnt_v, [iv], ones)
        plsc.addupdate_scatter(lsum_v, [iv], lab_v[sl])

    for j in range(M // L):
        sl = pl.ds(j * L, L)
        size = cnt_v[sl]
        denom = jnp.maximum(size, 1.0)
        mean_lab = lsum_v[sl] / denom
        mask = jnp.where(mean_lab > 0.5, 1.0, 0.0).astype(jnp.float32)
        scale_v[sl] = mask / denom
        obuf_a[0, sl] = size * mask

    @pl.when(dbase == 0)
    def _():
        pltpu.sync_copy(obuf_a.at[0], outs_hbm.at[k])

    def process(buf, obuf, osem, d, first):
        accs = (acc0_v, acc1_v, acc2_v, acc3_v)
        for acc in accs:
            for j in range(M // L):
                acc[pl.ds(j * L, L)] = zeros

        @plsc.parallel_loop(0, GROUPS, 1, unroll=UNROLL)
        def _seg(i):
            sl = pl.ds(i * L, L)
            iv = idx_v[sl]
            for r in range(R):
                plsc.addupdate_scatter(accs[r], [iv], buf[r, sl])

        @pl.when(jnp.logical_not(first))
        def _():
            pltpu.make_async_copy(obuf, outf_hbm.at[k, pl.ds(d, R)], osem).wait()

        for r in range(R):
            for j in range(M // L):
                sl = pl.ds(j * L, L)
                obuf[r, sl] = accs[r][sl] * scale_v[sl]
        pltpu.async_copy(obuf, outf_hbm.at[k, pl.ds(d, R)], osem)

    def quad_step(j, _):
        d0 = dbase + 2 * R * j
        d1 = d0 + R
        pltpu.make_async_copy(feat_hbm.at[k, pl.ds(d0, R)], buf_a, sem_a).wait()
        pltpu.async_copy(feat_hbm.at[k, pl.ds(d1, R)], buf_b, sem_b)
        process(buf_a, obuf_a, osem_a, d0, j == 0)
        pltpu.make_async_copy(feat_hbm.at[k, pl.ds(d1, R)], buf_b, sem_b).wait()

        @pl.when(j < HALF - 1)
        def _():
            pltpu.async_copy(feat_hbm.at[k, pl.ds(d1 + R, R)], buf_a, sem_a)

        process(buf_b, obuf_b, osem_b, d1, j == 0)
        return 0

    lax.fori_loop(0, HALF, quad_step, 0)

    dlast = dbase + D_PER_W - R
    pltpu.make_async_copy(obuf_a, outf_hbm.at[k, pl.ds(dlast, R)], osem_a).wait()
    pltpu.make_async_copy(obuf_b, outf_hbm.at[k, pl.ds(dlast, R)], osem_b).wait()


@jax.jit
def kernel(feat, label, cluster_label):
    mesh = plsc.VectorSubcoreMesh(core_axis_name="c", subcore_axis_name="s")
    fn = functools.partial(
        pl.kernel,
        out_type=[
            jax.ShapeDtypeStruct((K, D, M), jnp.float32),
            jax.ShapeDtypeStruct((K, M), jnp.float32),
        ],
        mesh=mesh,
        compiler_params=pltpu.CompilerParams(
            needs_layout_passes=False,
            disable_bounds_checks=True,
        ),
        scratch_types=[
            pltpu.VMEM((N,), jnp.int32),
            pltpu.VMEM((N,), jnp.float32),
            pltpu.VMEM((R, N), jnp.float32),
            pltpu.VMEM((R, N), jnp.float32),
            pltpu.VMEM((M,), jnp.float32),
            pltpu.VMEM((M,), jnp.float32),
            pltpu.VMEM((M,), jnp.float32),
            pltpu.VMEM((M,), jnp.float32),
            pltpu.VMEM((M,), jnp.float32),
            pltpu.VMEM((M,), jnp.float32),
            pltpu.VMEM((M,), jnp.float32),
            pltpu.VMEM((R, M), jnp.float32),
            pltpu.VMEM((R, M), jnp.float32),
            pltpu.SemaphoreType.DMA,
            pltpu.SemaphoreType.DMA,
            pltpu.SemaphoreType.DMA,
            pltpu.SemaphoreType.DMA,
            pltpu.SemaphoreType.DMA,
        ],
    )(_sc_body)
    outf, outs = fn(feat, label, cluster_label)
    return outf, outs
